# baseline (speedup 1.0000x reference)
import functools

import jax
import jax.numpy as jnp
import numpy as np
from jax import lax
from jax.experimental import pallas as pl
from jax.experimental.pallas import tpu as pltpu
from jax.experimental.pallas import tpu_sc as plsc

N = 10000
E = 320000
P = 30000
IN_CH = 9
LATENT = 128
K = 512
LEVELS = 3
COMMIT = 0.25

CPAD = 16
NW = 32
EPW = E // NW
C = 80
NCH = EPW // C
NPAD = 10240
RPT = NPAD // 16
NZCH = RPT // C

BN = 1000
G = N // BN


def _make_sc_agg(W):
    mesh = plsc.VectorSubcoreMesh(core_axis_name="c", subcore_axis_name="s")

    def body(table, src3, dst3, out, src_i, dst_i, rows_a, rows_b, acc_sh,
             sem_a, sem_b, sems_a, sems_b):
        c = lax.axis_index("c")
        s = lax.axis_index("s")
        wid = s * 2 + c

        pltpu.sync_copy(src3.at[wid], src_i)
        pltpu.sync_copy(dst3.at[wid], dst_i)
        pltpu.async_copy(table.at[src_i.at[0]], rows_a, sem_a)
        pltpu.async_copy(table.at[src_i.at[1]], rows_b, sem_b)

        zvec = jnp.zeros((16,), jnp.float32)

        def zrow(r, carry):
            for j in range(W // 16):
                rows_b[r, pl.ds(j * 16, 16)] = zvec
            return carry

        lax.fori_loop(0, C, zrow, 0)
        for k2 in range(NZCH):
            pltpu.sync_copy(rows_b, acc_sh.at[pl.ds(s * RPT + k2 * C, C)])
        plsc.subcore_barrier()

        def ebody(p, carry):
            j0 = 2 * p
            pltpu.make_async_copy(table.at[src_i.at[j0]], rows_a, sem_a).wait()
            pltpu.async_copy(rows_a, acc_sh.at[dst_i.at[j0]], sems_a, add=True)
            pltpu.make_async_copy(table.at[src_i.at[j0 + 1]], rows_b, sem_b).wait()
            pltpu.async_copy(rows_b, acc_sh.at[dst_i.at[j0 + 1]], sems_b, add=True)
            pltpu.make_async_copy(rows_a, acc_sh.at[dst_i.at[j0]], sems_a).wait()

            @pl.when(j0 + 2 < NCH)
            def _():
                pltpu.async_copy(table.at[src_i.at[j0 + 2]], rows_a, sem_a)

            pltpu.make_async_copy(rows_b, acc_sh.at[dst_i.at[j0 + 1]], sems_b).wait()

            @pl.when(j0 + 3 < NCH)
            def _():
                pltpu.async_copy(table.at[src_i.at[j0 + 3]], rows_b, sem_b)

            return carry

        lax.fori_loop(0, (NCH - 1) // 2, ebody, 0)
        pltpu.make_async_copy(table.at[src_i.at[NCH - 1]], rows_a, sem_a).wait()
        pltpu.sync_copy(rows_a, acc_sh.at[dst_i.at[NCH - 1]], add=True)
        plsc.subcore_barrier()

        for k2 in range(NZCH):
            r0 = s * RPT + k2 * C
            pltpu.sync_copy(acc_sh.at[pl.ds(r0, C)], rows_a)
            pltpu.sync_copy(rows_a, out.at[pl.ds(c * NPAD + r0, C)])

    return pl.kernel(
        body,
        out_type=jax.ShapeDtypeStruct((2 * NPAD, W), jnp.float32),
        mesh=mesh,
        compiler_params=pltpu.CompilerParams(use_tc_tiling_on_sc=False),
        scratch_types=[
            pltpu.VMEM((NCH, C), jnp.int32),
            pltpu.VMEM((NCH, C), jnp.int32),
            pltpu.VMEM((C, W), jnp.float32),
            pltpu.VMEM((C, W), jnp.float32),
            pltpu.VMEM_SHARED((NPAD, W), jnp.float32),
            pltpu.SemaphoreType.DMA,
            pltpu.SemaphoreType.DMA,
            pltpu.SemaphoreType.DMA,
            pltpu.SemaphoreType.DMA,
        ],
    )


CC = 80
NCHUNK = P // CC


def _sc_cons(rflat, ta, la, tb, lb):
    mesh = plsc.VectorSubcoreMesh(core_axis_name="c", subcore_axis_name="s")

    def body(rflat_h, ta_h, la_h, tb_h, lb_h, out, tab_v, ta_v, la_v, tb_v, lb_v, scr_v):
        c = lax.axis_index("c")
        s = lax.axis_index("s")
        wid = s * 2 + c
        pltpu.sync_copy(rflat_h, tab_v)
        n = (NCHUNK - wid + NW - 1) // NW

        def chunk(j, acc):
            base = (wid + NW * j) * CC
            pltpu.sync_copy(ta_h.at[pl.ds(base, CC)], ta_v)
            pltpu.sync_copy(la_h.at[pl.ds(base, CC)], la_v)
            pltpu.sync_copy(tb_h.at[pl.ds(base, CC)], tb_v)
            pltpu.sync_copy(lb_h.at[pl.ds(base, CC)], lb_v)
            for g in range(CC // 16):
                tav = ta_v[pl.ds(g * 16, 16)]
                lav = la_v[pl.ds(g * 16, 16)]
                tbv = tb_v[pl.ds(g * 16, 16)]
                lbv = lb_v[pl.ds(g * 16, 16)]
                fa = tav * 9 + lav * 3
                fb = tbv * 9 + lbv * 3
                for cc in range(3):
                    va = plsc.load_gather(tab_v, [fa + cc])
                    vb = plsc.load_gather(tab_v, [fb + cc])
                    d = va - vb
                    acc = acc + d * d
            return acc

        acc = lax.fori_loop(0, n, chunk, jnp.zeros((16,), jnp.float32))
        scr_v[...] = acc
        pltpu.sync_copy(scr_v, out.at[pl.ds(wid * 16, 16)])

    k = pl.kernel(
        body,
        out_type=jax.ShapeDtypeStruct((NW * 16,), jnp.float32),
        mesh=mesh,
        compiler_params=pltpu.CompilerParams(needs_layout_passes=False),
        scratch_types=[
            pltpu.VMEM((N * 9,), jnp.float32),
            pltpu.VMEM((CC,), jnp.int32),
            pltpu.VMEM((CC,), jnp.int32),
            pltpu.VMEM((CC,), jnp.int32),
            pltpu.VMEM((CC,), jnp.int32),
            pltpu.VMEM((16,), jnp.float32),
        ],
    )
    return k(rflat, ta, la, tb, lb)


def _tc1_body(a0, a1, x, w, b, h_ref, dinv_ref):
    agg = a0[...] + a1[...]
    deg = jnp.clip(agg[:, 9:10], 1.0, None)
    dinv = 1.0 / deg
    comb = x[...] + agg * dinv
    h = jnp.dot(comb, w[...], preferred_element_type=jnp.float32) + b[...]
    h_ref[...] = jnp.maximum(h, 0.0)
    dinv_ref[...] = dinv


def _tc_comb_body(a0, a1, x, dinv, w, b, o_ref, *, relu):
    agg = a0[...] + a1[...]
    comb = x[...] + agg * dinv[...]
    o = jnp.dot(comb, w[...], preferred_element_type=jnp.float32) + b[...]
    o_ref[...] = jnp.maximum(o, 0.0) if relu else o


def _tc_mm_body(x, w, o_ref):
    o_ref[...] = jnp.dot(x[...], w[...], preferred_element_type=jnp.float32)


def _tc_rvq_body(z, cb, zq_ref, idx_ref, loss_ref):
    i = pl.program_id(0)

    @pl.when(i == 0)
    def _():
        loss_ref[...] = jnp.zeros_like(loss_ref)

    res = z[...]
    zq = jnp.zeros_like(res)
    iota = lax.broadcasted_iota(jnp.int32, (1, K), 1)
    lsum = jnp.float32(0.0)
    idx_cols = []
    for l in range(LEVELS):
        cbl = cb[l]
        cn = jnp.sum(cbl * cbl, axis=1)[None, :]
        rn = jnp.sum(res * res, axis=1, keepdims=True)
        d = (rn - 2.0 * lax.dot_general(res, cbl, (((1,), (1,)), ((), ())),
                                        preferred_element_type=jnp.float32) + cn)
        mv = jnp.min(d, axis=1, keepdims=True)
        idxv = jnp.min(jnp.where(d == mv, iota, K), axis=1, keepdims=True)
        one = (iota == idxv).astype(jnp.float32)
        cb_hi = cbl.astype(jnp.bfloat16).astype(jnp.float32)
        rem1 = cbl - cb_hi
        cb_mid = rem1.astype(jnp.bfloat16).astype(jnp.float32)
        cb_lo = rem1 - cb_mid
        q = (jnp.dot(one, cb_hi, preferred_element_type=jnp.float32)
             + jnp.dot(one, cb_mid, preferred_element_type=jnp.float32)
             + jnp.dot(one, cb_lo, preferred_element_type=jnp.float32))
        lsum = lsum + jnp.sum((res - q) ** 2)
        zq = zq + q
        res = res - q
        idx_cols.append(idxv)
    zq_ref[...] = zq
    pad = jnp.zeros((idxv.shape[0], 8 - LEVELS), jnp.int32)
    idx_ref[...] = jnp.concatenate(idx_cols + [pad], axis=1)
    loss_ref[...] = loss_ref[...] + jnp.reshape(lsum, (1, 1))


def _tc_fin_body(u, a0, a1, dinv, b4p, y, p1m, p2m, r_ref, rl_ref):
    i = pl.program_id(0)

    @pl.when(i == 0)
    def _():
        rl_ref[...] = jnp.zeros_like(rl_ref)

    agg = a0[...] + a1[...]
    r = u[...] + agg * dinv[...] + b4p[...]
    r_ref[...] = r
    t = y[...]
    t1 = jnp.dot(t, p1m[...], preferred_element_type=jnp.float32)
    t2 = jnp.dot(t, p2m[...], preferred_element_type=jnp.float32)
    p0 = jnp.sum(jnp.abs(r - t), axis=1) / 9.0
    p1 = jnp.sum(jnp.abs(r - t1), axis=1) / 9.0
    p2 = jnp.sum(jnp.abs(r - t2), axis=1) / 9.0
    m = jnp.minimum(p0, jnp.minimum(p1, p2))
    rl_ref[...] = rl_ref[...] + jnp.reshape(jnp.sum(m), (1, 1))


def _tc_scalars_body(rl, vq, parts, rl_o, vq_o, c_o, t_o):
    rlv = rl[...] / N
    vqv = vq[...] * (1.0 + COMMIT) / (N * LATENT)
    cv = jnp.reshape(jnp.sum(parts[...]), (1, 1)) / (3.0 * P)
    rl_o[...] = rlv
    vq_o[...] = vqv
    c_o[...] = cv
    t_o[...] = rlv + vqv + 0.3 * cv


def _row_spec(w):
    return pl.BlockSpec((BN, w), lambda i: (i, 0))


def _full_spec(shape):
    nd = len(shape)
    return pl.BlockSpec(shape, lambda i, _nd=nd: (0,) * _nd)


def _scalar_spec():
    return pl.BlockSpec((1, 1), lambda i: (0, 0))


_P1 = np.zeros((CPAD, CPAD), np.float32)
_P2 = np.zeros((CPAD, CPAD), np.float32)
for _j in range(9):
    _P1[(_j + 3) % 9, _j] = 1.0
    _P2[(_j + 6) % 9, _j] = 1.0
for _j in range(9, CPAD):
    _P1[_j, _j] = 1.0
    _P2[_j, _j] = 1.0


def kernel(x, edge_index, y, sv_tri_a, sv_local_a, sv_tri_b, sv_local_b,
           W1, b1, W2, b2, W3, b3, W4, b4, codebooks):
    src = edge_index[0].reshape(NW, NCH, C)
    dst = edge_index[1].reshape(NW, NCH, C)

    x16 = jnp.concatenate(
        [x, jnp.ones((N, 1), jnp.float32), jnp.zeros((N, CPAD - IN_CH - 1), jnp.float32)], axis=1)
    W1p = jnp.concatenate([W1, jnp.zeros((CPAD - IN_CH, LATENT), jnp.float32)], axis=0)
    W4p = jnp.concatenate([W4, jnp.zeros((LATENT, CPAD - IN_CH), jnp.float32)], axis=1)
    b4p = jnp.concatenate([b4, jnp.zeros((CPAD - IN_CH,), jnp.float32)])[None, :]
    y16 = jnp.concatenate([y, jnp.zeros((N, CPAD - IN_CH), jnp.float32)], axis=1)
    b1r = b1[None, :]
    b2r = b2[None, :]
    b3r = b3[None, :]
    p1m = jnp.asarray(_P1)
    p2m = jnp.asarray(_P2)

    agg16 = _make_sc_agg(CPAD)(x16, src, dst)
    a0, a1 = agg16[:N], agg16[NPAD:NPAD + N]

    h, dinv = pl.pallas_call(
        _tc1_body,
        grid=(G,),
        in_specs=[_row_spec(CPAD), _row_spec(CPAD), _row_spec(CPAD),
                  _full_spec((CPAD, LATENT)), _full_spec((1, LATENT))],
        out_specs=[_row_spec(LATENT), _row_spec(1)],
        out_shape=[jax.ShapeDtypeStruct((N, LATENT), jnp.float32),
                   jax.ShapeDtypeStruct((N, 1), jnp.float32)],
    )(a0, a1, x16, W1p, b1r)

    aggh = _make_sc_agg(LATENT)(h, src, dst)

    z_e = pl.pallas_call(
        functools.partial(_tc_comb_body, relu=False),
        grid=(G,),
        in_specs=[_row_spec(LATENT), _row_spec(LATENT), _row_spec(LATENT), _row_spec(1),
                  _full_spec((LATENT, LATENT)), _full_spec((1, LATENT))],
        out_specs=_row_spec(LATENT),
        out_shape=jax.ShapeDtypeStruct((N, LATENT), jnp.float32),
    )(aggh[:N], aggh[NPAD:NPAD + N], h, dinv, W2, b2r)

    z_q, idx8, vqsum = pl.pallas_call(
        _tc_rvq_body,
        grid=(G,),
        in_specs=[_row_spec(LATENT), _full_spec((LEVELS, K, LATENT))],
        out_specs=[_row_spec(LATENT), _row_spec(8), _scalar_spec()],
        out_shape=[jax.ShapeDtypeStruct((N, LATENT), jnp.float32),
                   jax.ShapeDtypeStruct((N, 8), jnp.int32),
                   jax.ShapeDtypeStruct((1, 1), jnp.float32)],
    )(z_e, codebooks)

    aggq = _make_sc_agg(LATENT)(z_q, src, dst)

    h2 = pl.pallas_call(
        functools.partial(_tc_comb_body, relu=True),
        grid=(G,),
        in_specs=[_row_spec(LATENT), _row_spec(LATENT), _row_spec(LATENT), _row_spec(1),
                  _full_spec((LATENT, LATENT)), _full_spec((1, LATENT))],
        out_specs=_row_spec(LATENT),
        out_shape=jax.ShapeDtypeStruct((N, LATENT), jnp.float32),
    )(aggq[:N], aggq[NPAD:NPAD + N], z_q, dinv, W3, b3r)

    u16 = pl.pallas_call(
        _tc_mm_body,
        grid=(G,),
        in_specs=[_row_spec(LATENT), _full_spec((LATENT, CPAD))],
        out_specs=_row_spec(CPAD),
        out_shape=jax.ShapeDtypeStruct((N, CPAD), jnp.float32),
    )(h2, W4p)

    aggu = _make_sc_agg(CPAD)(u16, src, dst)

    r16, rlsum = pl.pallas_call(
        _tc_fin_body,
        grid=(G,),
        in_specs=[_row_spec(CPAD), _row_spec(CPAD), _row_spec(CPAD), _row_spec(1),
                  _full_spec((1, CPAD)), _row_spec(CPAD),
                  _full_spec((CPAD, CPAD)), _full_spec((CPAD, CPAD))],
        out_specs=[_row_spec(CPAD), _scalar_spec()],
        out_shape=[jax.ShapeDtypeStruct((N, CPAD), jnp.float32),
                   jax.ShapeDtypeStruct((1, 1), jnp.float32)],
    )(u16, aggu[:N], aggu[NPAD:NPAD + N], dinv, b4p, y16, p1m, p2m)

    recon = r16[:, :IN_CH]
    rflat = recon.reshape(-1)

    parts = _sc_cons(rflat, sv_tri_a, sv_local_a, sv_tri_b, sv_local_b)

    rl_o, vq_o, c_o, t_o = pl.pallas_call(
        _tc_scalars_body,
        grid=(1,),
        in_specs=[_scalar_spec(), _scalar_spec(), _full_spec((NW, 16))],
        out_specs=[_scalar_spec(), _scalar_spec(), _scalar_spec(), _scalar_spec()],
        out_shape=[jax.ShapeDtypeStruct((1, 1), jnp.float32)] * 4,
    )(rlsum, vqsum, parts.reshape(NW, 16))

    indices = idx8[:, :LEVELS]
    return (recon, rl_o[0, 0], vq_o[0, 0], c_o[0, 0], t_o[0, 0], indices, z_e, z_q)


# speedup vs baseline: 1.1748x; 1.1748x over previous
import jax
import jax.numpy as jnp
import numpy as np
from jax import lax
from jax.experimental import pallas as pl
from jax.experimental.pallas import tpu as pltpu
from jax.experimental.pallas import tpu_sc as plsc

N = 10000
E = 320000
P = 30000
IN_CH = 9
LATENT = 128
K = 512
LEVELS = 3
COMMIT = 0.25

CPAD = 16
NW = 32
EPW = E // NW
C = 80
NCH = EPW // C
NPAD = 10240
RPT = NPAD // 16
NZCH = RPT // C

BN = 1000
G = N // BN


def _make_sc_agg(W):
    mesh = plsc.VectorSubcoreMesh(core_axis_name="c", subcore_axis_name="s")

    def body(table, src3, dst3, out, src_i, dst_i, rows_a, rows_b, acc_sh,
             sem_a, sem_b):
        c = lax.axis_index("c")
        s = lax.axis_index("s")
        wid = s * 2 + c

        pltpu.sync_copy(src3.at[wid], src_i)
        pltpu.sync_copy(dst3.at[wid], dst_i)
        pltpu.async_copy(table.at[src_i.at[0]], rows_a, sem_a)

        zvec = jnp.zeros((16,), jnp.float32)

        def zrow(r, carry):
            for j in range(W // 16):
                rows_b[r, pl.ds(j * 16, 16)] = zvec
            return carry

        lax.fori_loop(0, C, zrow, 0)
        for k2 in range(NZCH):
            pltpu.sync_copy(rows_b, acc_sh.at[pl.ds(s * RPT + k2 * C, C)])
        plsc.subcore_barrier()

        def ebody(p, carry):
            j0 = 2 * p
            pltpu.make_async_copy(table.at[src_i.at[j0]], rows_a, sem_a).wait()
            pltpu.async_copy(table.at[src_i.at[j0 + 1]], rows_b, sem_b)
            pltpu.sync_copy(rows_a, acc_sh.at[dst_i.at[j0]], add=True)
            pltpu.async_copy(table.at[src_i.at[j0 + 2]], rows_a, sem_a)
            pltpu.make_async_copy(table.at[src_i.at[j0 + 1]], rows_b, sem_b).wait()
            pltpu.sync_copy(rows_b, acc_sh.at[dst_i.at[j0 + 1]], add=True)
            return carry

        lax.fori_loop(0, (NCH - 1) // 2, ebody, 0)
        pltpu.make_async_copy(table.at[src_i.at[NCH - 1]], rows_a, sem_a).wait()
        pltpu.sync_copy(rows_a, acc_sh.at[dst_i.at[NCH - 1]], add=True)
        plsc.subcore_barrier()

        for k2 in range(NZCH):
            r0 = s * RPT + k2 * C
            pltpu.sync_copy(acc_sh.at[pl.ds(r0, C)], rows_a)
            pltpu.sync_copy(rows_a, out.at[pl.ds(c * NPAD + r0, C)])

    return pl.kernel(
        body,
        out_type=jax.ShapeDtypeStruct((2 * NPAD, W), jnp.float32),
        mesh=mesh,
        compiler_params=pltpu.CompilerParams(use_tc_tiling_on_sc=False),
        scratch_types=[
            pltpu.VMEM((NCH, C), jnp.int32),
            pltpu.VMEM((NCH, C), jnp.int32),
            pltpu.VMEM((C, W), jnp.float32),
            pltpu.VMEM((C, W), jnp.float32),
            pltpu.VMEM_SHARED((NPAD, W), jnp.float32),
            pltpu.SemaphoreType.DMA,
            pltpu.SemaphoreType.DMA,
        ],
    )


PPAD = 30720
PPW = PPAD // 16


def _sc_cons(rflat, ta, la, tb, lb, rlvq16):
    mesh = plsc.VectorSubcoreMesh(core_axis_name="c", subcore_axis_name="s")

    def body(rflat_h, ta_h, la_h, tb_h, lb_h, rlvq_h, out,
             tab_v, ta_v, la_v, tb_v, lb_v, scr_v, red_v, rlvq_v, stage_sh):
        c = lax.axis_index("c")
        s = lax.axis_index("s")

        @pl.when(c == 0)
        def _():
            pltpu.sync_copy(rflat_h, tab_v)
            pltpu.sync_copy(ta_h.at[s], ta_v)
            pltpu.sync_copy(la_h.at[s], la_v)
            pltpu.sync_copy(tb_h.at[s], tb_v)
            pltpu.sync_copy(lb_h.at[s], lb_v)

            def chunk(g, acc):
                tav = ta_v[pl.ds(g * 16, 16)]
                lav = la_v[pl.ds(g * 16, 16)]
                tbv = tb_v[pl.ds(g * 16, 16)]
                lbv = lb_v[pl.ds(g * 16, 16)]
                fa = tav * 9 + lav * 3
                fb = tbv * 9 + lbv * 3
                for cc in range(3):
                    va = plsc.load_gather(tab_v, [fa + cc])
                    vb = plsc.load_gather(tab_v, [fb + cc])
                    d = va - vb
                    acc = acc + d * d
                return acc

            acc = lax.fori_loop(0, PPW // 16, chunk, jnp.zeros((16,), jnp.float32))
            scr_v[...] = acc
            pltpu.sync_copy(scr_v, stage_sh.at[s])

        plsc.subcore_barrier()

        @pl.when((c == 0) & (s == 0))
        def _():
            pltpu.sync_copy(stage_sh, red_v)
            pltpu.sync_copy(rlvq_h, rlvq_v)

            def red(i, acc):
                return acc + red_v[i]

            tot = lax.fori_loop(0, 16, red, jnp.zeros((16,), jnp.float32))
            cons_scalar = jnp.sum(tot) * (1.0 / (3.0 * P))
            consm = jnp.full((16,), cons_scalar, jnp.float32)
            zero16 = jnp.zeros((16,), jnp.int32)
            rlm = plsc.load_gather(rlvq_v, [zero16]) * (1.0 / N)
            vqm = plsc.load_gather(rlvq_v, [zero16 + 1]) * ((1.0 + COMMIT) / (N * LATENT))
            totm = rlm + vqm + 0.3 * consm
            iota = lax.iota(jnp.int32, 16)
            res = jnp.where(iota == 0, rlm,
                            jnp.where(iota == 1, vqm,
                                      jnp.where(iota == 2, consm, totm)))
            scr_v[...] = res
            pltpu.sync_copy(scr_v, out)

    k = pl.kernel(
        body,
        out_type=jax.ShapeDtypeStruct((16,), jnp.float32),
        mesh=mesh,
        compiler_params=pltpu.CompilerParams(needs_layout_passes=False),
        scratch_types=[
            pltpu.VMEM((N * 9,), jnp.float32),
            pltpu.VMEM((PPW,), jnp.int32),
            pltpu.VMEM((PPW,), jnp.int32),
            pltpu.VMEM((PPW,), jnp.int32),
            pltpu.VMEM((PPW,), jnp.int32),
            pltpu.VMEM((16,), jnp.float32),
            pltpu.VMEM((16, 16), jnp.float32),
            pltpu.VMEM((16,), jnp.float32),
            pltpu.VMEM_SHARED((16, 16), jnp.float32),
        ],
    )
    return k(rflat, ta, la, tb, lb, rlvq16)


def _tc1_body(a0, a1, x, w, b, h_ref, dinv_ref):
    agg = a0[...] + a1[...]
    deg = jnp.clip(agg[:, 9:10], 1.0, None)
    dinv = 1.0 / deg
    comb = x[...] + agg * dinv
    h = jnp.dot(comb, w[...], preferred_element_type=jnp.float32) + b[...]
    h_ref[...] = jnp.maximum(h, 0.0)
    dinv_ref[...] = dinv


def _tc3u_body(a0, a1, x, dinv, w3, b3, w4p, u_ref):
    agg = a0[...] + a1[...]
    comb = x[...] + agg * dinv[...]
    h2 = jnp.maximum(jnp.dot(comb, w3[...], preferred_element_type=jnp.float32) + b3[...], 0.0)
    u_ref[...] = jnp.dot(h2, w4p[...], preferred_element_type=jnp.float32)


def _tc_ze_rvq_body(a0, a1, h, dinv, w2, b2, cb, ze_ref, zq_ref, idx_ref, loss_ref):
    i = pl.program_id(0)

    @pl.when(i == 0)
    def _():
        loss_ref[...] = jnp.zeros_like(loss_ref)

    agg = a0[...] + a1[...]
    comb = h[...] + agg * dinv[...]
    z = jnp.dot(comb, w2[...], preferred_element_type=jnp.float32) + b2[...]
    ze_ref[...] = z
    res = z
    zq = jnp.zeros_like(res)
    iota = lax.broadcasted_iota(jnp.int32, (1, K), 1)
    lsum = jnp.float32(0.0)
    idx_cols = []
    for l in range(LEVELS):
        cbl = cb[l]
        cn = jnp.sum(cbl * cbl, axis=1)[None, :]
        rn = jnp.sum(res * res, axis=1, keepdims=True)
        d = (rn - 2.0 * lax.dot_general(res, cbl, (((1,), (1,)), ((), ())),
                                        preferred_element_type=jnp.float32) + cn)
        mv = jnp.min(d, axis=1, keepdims=True)
        idxv = jnp.min(jnp.where(d == mv, iota, K), axis=1, keepdims=True)
        one = (iota == idxv).astype(jnp.float32)
        cb_hi = cbl.astype(jnp.bfloat16).astype(jnp.float32)
        rem1 = cbl - cb_hi
        cb_mid = rem1.astype(jnp.bfloat16).astype(jnp.float32)
        cb_lo = rem1 - cb_mid
        q = (jnp.dot(one, cb_hi, preferred_element_type=jnp.float32)
             + jnp.dot(one, cb_mid, preferred_element_type=jnp.float32)
             + jnp.dot(one, cb_lo, preferred_element_type=jnp.float32))
        lsum = lsum + jnp.sum((res - q) ** 2)
        zq = zq + q
        res = res - q
        idx_cols.append(idxv)
    zq_ref[...] = zq
    pad = jnp.zeros((idxv.shape[0], 8 - LEVELS), jnp.int32)
    idx_ref[...] = jnp.concatenate(idx_cols + [pad], axis=1)
    loss_ref[...] = loss_ref[...] + jnp.reshape(lsum, (1, 1))


def _tc_fin_body(u, a0, a1, dinv, b4p, y, p1m, p2m, r_ref, rl_ref):
    i = pl.program_id(0)

    @pl.when(i == 0)
    def _():
        rl_ref[...] = jnp.zeros_like(rl_ref)

    agg = a0[...] + a1[...]
    r = u[...] + agg * dinv[...] + b4p[...]
    r_ref[...] = r
    t = y[...]
    t1 = jnp.dot(t, p1m[...], preferred_element_type=jnp.float32)
    t2 = jnp.dot(t, p2m[...], preferred_element_type=jnp.float32)
    p0 = jnp.sum(jnp.abs(r - t), axis=1) / 9.0
    p1 = jnp.sum(jnp.abs(r - t1), axis=1) / 9.0
    p2 = jnp.sum(jnp.abs(r - t2), axis=1) / 9.0
    m = jnp.minimum(p0, jnp.minimum(p1, p2))
    rl_ref[...] = rl_ref[...] + jnp.reshape(jnp.sum(m), (1, 1))


def _row_spec(w):
    return pl.BlockSpec((BN, w), lambda i: (i, 0))


def _full_spec(shape):
    nd = len(shape)
    return pl.BlockSpec(shape, lambda i, _nd=nd: (0,) * _nd)


def _scalar_spec():
    return pl.BlockSpec((1, 1), lambda i: (0, 0))


_P1 = np.zeros((CPAD, CPAD), np.float32)
_P2 = np.zeros((CPAD, CPAD), np.float32)
for _j in range(9):
    _P1[(_j + 3) % 9, _j] = 1.0
    _P2[(_j + 6) % 9, _j] = 1.0
for _j in range(9, CPAD):
    _P1[_j, _j] = 1.0
    _P2[_j, _j] = 1.0


def kernel(x, edge_index, y, sv_tri_a, sv_local_a, sv_tri_b, sv_local_b,
           W1, b1, W2, b2, W3, b3, W4, b4, codebooks):
    src = edge_index[0].reshape(NW, NCH, C)
    dst = edge_index[1].reshape(NW, NCH, C)

    x16 = jnp.concatenate(
        [x, jnp.ones((N, 1), jnp.float32), jnp.zeros((N, CPAD - IN_CH - 1), jnp.float32)], axis=1)
    W1p = jnp.concatenate([W1, jnp.zeros((CPAD - IN_CH, LATENT), jnp.float32)], axis=0)
    W4p = jnp.concatenate([W4, jnp.zeros((LATENT, CPAD - IN_CH), jnp.float32)], axis=1)
    b4p = jnp.concatenate([b4, jnp.zeros((CPAD - IN_CH,), jnp.float32)])[None, :]
    y16 = jnp.concatenate([y, jnp.zeros((N, CPAD - IN_CH), jnp.float32)], axis=1)
    b1r = b1[None, :]
    b2r = b2[None, :]
    b3r = b3[None, :]
    p1m = jnp.asarray(_P1)
    p2m = jnp.asarray(_P2)

    agg16 = _make_sc_agg(CPAD)(x16, src, dst)
    a0, a1 = agg16[:N], agg16[NPAD:NPAD + N]

    h, dinv = pl.pallas_call(
        _tc1_body,
        grid=(G,),
        in_specs=[_row_spec(CPAD), _row_spec(CPAD), _row_spec(CPAD),
                  _full_spec((CPAD, LATENT)), _full_spec((1, LATENT))],
        out_specs=[_row_spec(LATENT), _row_spec(1)],
        out_shape=[jax.ShapeDtypeStruct((N, LATENT), jnp.float32),
                   jax.ShapeDtypeStruct((N, 1), jnp.float32)],
    )(a0, a1, x16, W1p, b1r)

    aggh = _make_sc_agg(LATENT)(h, src, dst)

    z_e, z_q, idx8, vqsum = pl.pallas_call(
        _tc_ze_rvq_body,
        grid=(G,),
        in_specs=[_row_spec(LATENT), _row_spec(LATENT), _row_spec(LATENT), _row_spec(1),
                  _full_spec((LATENT, LATENT)), _full_spec((1, LATENT)),
                  _full_spec((LEVELS, K, LATENT))],
        out_specs=[_row_spec(LATENT), _row_spec(LATENT), _row_spec(8), _scalar_spec()],
        out_shape=[jax.ShapeDtypeStruct((N, LATENT), jnp.float32),
                   jax.ShapeDtypeStruct((N, LATENT), jnp.float32),
                   jax.ShapeDtypeStruct((N, 8), jnp.int32),
                   jax.ShapeDtypeStruct((1, 1), jnp.float32)],
    )(aggh[:N], aggh[NPAD:NPAD + N], h, dinv, W2, b2r, codebooks)

    aggq = _make_sc_agg(LATENT)(z_q, src, dst)

    u16 = pl.pallas_call(
        _tc3u_body,
        grid=(G,),
        in_specs=[_row_spec(LATENT), _row_spec(LATENT), _row_spec(LATENT), _row_spec(1),
                  _full_spec((LATENT, LATENT)), _full_spec((1, LATENT)),
                  _full_spec((LATENT, CPAD))],
        out_specs=_row_spec(CPAD),
        out_shape=jax.ShapeDtypeStruct((N, CPAD), jnp.float32),
    )(aggq[:N], aggq[NPAD:NPAD + N], z_q, dinv, W3, b3r, W4p)

    aggu = _make_sc_agg(CPAD)(u16, src, dst)

    r16, rlsum = pl.pallas_call(
        _tc_fin_body,
        grid=(G,),
        in_specs=[_row_spec(CPAD), _row_spec(CPAD), _row_spec(CPAD), _row_spec(1),
                  _full_spec((1, CPAD)), _row_spec(CPAD),
                  _full_spec((CPAD, CPAD)), _full_spec((CPAD, CPAD))],
        out_specs=[_row_spec(CPAD), _scalar_spec()],
        out_shape=[jax.ShapeDtypeStruct((N, CPAD), jnp.float32),
                   jax.ShapeDtypeStruct((1, 1), jnp.float32)],
    )(u16, aggu[:N], aggu[NPAD:NPAD + N], dinv, b4p, y16, p1m, p2m)

    recon = r16[:, :IN_CH]
    rflat = recon.reshape(-1)

    zp = jnp.zeros((PPAD - P,), jnp.int32)
    rlvq16 = jnp.concatenate([rlsum.reshape(1), vqsum.reshape(1),
                              jnp.zeros((14,), jnp.float32)])
    scal = _sc_cons(rflat,
                    jnp.concatenate([sv_tri_a, zp]).reshape(16, PPW),
                    jnp.concatenate([sv_local_a, zp]).reshape(16, PPW),
                    jnp.concatenate([sv_tri_b, zp]).reshape(16, PPW),
                    jnp.concatenate([sv_local_b, zp]).reshape(16, PPW),
                    rlvq16)

    indices = idx8[:, :LEVELS]
    return (recon, scal[0], scal[1], scal[2], scal[3], indices, z_e, z_q)


# speedup vs baseline: 1.1776x; 1.0024x over previous
import jax
import jax.numpy as jnp
import numpy as np
from jax import lax
from jax.experimental import pallas as pl
from jax.experimental.pallas import tpu as pltpu
from jax.experimental.pallas import tpu_sc as plsc

N = 10000
E = 320000
P = 30000
IN_CH = 9
LATENT = 128
K = 512
LEVELS = 3
COMMIT = 0.25

CPAD = 16
NW = 32
EPW = E // NW
C = 80
NCH = EPW // C
NPAD = 10240
RPT = NPAD // 16
NZCH = RPT // C

BN = 1000
G = N // BN


def _make_sc_agg(W):
    mesh = plsc.VectorSubcoreMesh(core_axis_name="c", subcore_axis_name="s")

    def body(table, src3, dst3, out, src_i, dst_i, rows_a, rows_b, acc_sh,
             sem_a, sem_b):
        c = lax.axis_index("c")
        s = lax.axis_index("s")
        wid = s * 2 + c

        pltpu.sync_copy(src3.at[wid], src_i)
        pltpu.sync_copy(dst3.at[wid], dst_i)
        pltpu.async_copy(table.at[src_i.at[0]], rows_a, sem_a)

        zvec = jnp.zeros((16,), jnp.float32)

        def zrow(r, carry):
            for j in range(W // 16):
                rows_b[r, pl.ds(j * 16, 16)] = zvec
            return carry

        lax.fori_loop(0, C, zrow, 0)
        for k2 in range(NZCH):
            pltpu.sync_copy(rows_b, acc_sh.at[pl.ds(s * RPT + k2 * C, C)])
        plsc.subcore_barrier()

        def ebody(p, carry):
            j0 = 2 * p
            pltpu.make_async_copy(table.at[src_i.at[j0]], rows_a, sem_a).wait()
            pltpu.async_copy(table.at[src_i.at[j0 + 1]], rows_b, sem_b)
            pltpu.sync_copy(rows_a, acc_sh.at[dst_i.at[j0]], add=True)
            pltpu.async_copy(table.at[src_i.at[j0 + 2]], rows_a, sem_a)
            pltpu.make_async_copy(table.at[src_i.at[j0 + 1]], rows_b, sem_b).wait()
            pltpu.sync_copy(rows_b, acc_sh.at[dst_i.at[j0 + 1]], add=True)
            return carry

        lax.fori_loop(0, (NCH - 1) // 2, ebody, 0)
        pltpu.make_async_copy(table.at[src_i.at[NCH - 1]], rows_a, sem_a).wait()
        pltpu.sync_copy(rows_a, acc_sh.at[dst_i.at[NCH - 1]], add=True)
        plsc.subcore_barrier()

        for k2 in range(NZCH):
            r0 = s * RPT + k2 * C
            pltpu.sync_copy(acc_sh.at[pl.ds(r0, C)], rows_a)
            pltpu.sync_copy(rows_a, out.at[pl.ds(c * NPAD + r0, C)])

    return pl.kernel(
        body,
        out_type=jax.ShapeDtypeStruct((2 * NPAD, W), jnp.float32),
        mesh=mesh,
        compiler_params=pltpu.CompilerParams(use_tc_tiling_on_sc=False),
        scratch_types=[
            pltpu.VMEM((NCH, C), jnp.int32),
            pltpu.VMEM((NCH, C), jnp.int32),
            pltpu.VMEM((C, W), jnp.float32),
            pltpu.VMEM((C, W), jnp.float32),
            pltpu.VMEM_SHARED((NPAD, W), jnp.float32),
            pltpu.SemaphoreType.DMA,
            pltpu.SemaphoreType.DMA,
        ],
    )


PPAD = 30720
PPW = PPAD // 16


def _sc_cons(rflat, ta, la, tb, lb):
    mesh = plsc.VectorSubcoreMesh(core_axis_name="c", subcore_axis_name="s")

    def body(rflat_h, ta_h, la_h, tb_h, lb_h, out, tab_v, ta_v, la_v, tb_v, lb_v, scr_v):
        c = lax.axis_index("c")
        s = lax.axis_index("s")

        @pl.when(c == 0)
        def _():
            pltpu.sync_copy(rflat_h, tab_v)
            pltpu.sync_copy(ta_h.at[s], ta_v)
            pltpu.sync_copy(la_h.at[s], la_v)
            pltpu.sync_copy(tb_h.at[s], tb_v)
            pltpu.sync_copy(lb_h.at[s], lb_v)

            def chunk(g, acc):
                tav = ta_v[pl.ds(g * 16, 16)]
                lav = la_v[pl.ds(g * 16, 16)]
                tbv = tb_v[pl.ds(g * 16, 16)]
                lbv = lb_v[pl.ds(g * 16, 16)]
                fa = tav * 9 + lav * 3
                fb = tbv * 9 + lbv * 3
                for cc in range(3):
                    va = plsc.load_gather(tab_v, [fa + cc])
                    vb = plsc.load_gather(tab_v, [fb + cc])
                    d = va - vb
                    acc = acc + d * d
                return acc

            acc = lax.fori_loop(0, PPW // 16, chunk, jnp.zeros((16,), jnp.float32))
            scr_v[...] = acc
            pltpu.sync_copy(scr_v, out.at[s])

    k = pl.kernel(
        body,
        out_type=jax.ShapeDtypeStruct((16, 16), jnp.float32),
        mesh=mesh,
        compiler_params=pltpu.CompilerParams(needs_layout_passes=False),
        scratch_types=[
            pltpu.VMEM((N * 9,), jnp.float32),
            pltpu.VMEM((PPW,), jnp.int32),
            pltpu.VMEM((PPW,), jnp.int32),
            pltpu.VMEM((PPW,), jnp.int32),
            pltpu.VMEM((PPW,), jnp.int32),
            pltpu.VMEM((16,), jnp.float32),
        ],
    )
    return k(rflat, ta, la, tb, lb)


def _tc_scalars_body(rl, vq, parts, rl_o, vq_o, c_o, t_o):
    rlv = rl[...] / N
    vqv = vq[...] * (1.0 + COMMIT) / (N * LATENT)
    cv = jnp.reshape(jnp.sum(parts[...]), (1, 1)) / (3.0 * P)
    rl_o[...] = rlv
    vq_o[...] = vqv
    c_o[...] = cv
    t_o[...] = rlv + vqv + 0.3 * cv


def _tc1_body(a0, a1, x, w, b, h_ref, dinv_ref):
    agg = a0[...] + a1[...]
    deg = jnp.clip(agg[:, 9:10], 1.0, None)
    dinv = 1.0 / deg
    comb = x[...] + agg * dinv
    h = jnp.dot(comb, w[...], preferred_element_type=jnp.float32) + b[...]
    h_ref[...] = jnp.maximum(h, 0.0)
    dinv_ref[...] = dinv


def _tc3u_body(a0, a1, x, dinv, w3, b3, w4p, u_ref):
    agg = a0[...] + a1[...]
    comb = x[...] + agg * dinv[...]
    h2 = jnp.maximum(jnp.dot(comb, w3[...], preferred_element_type=jnp.float32) + b3[...], 0.0)
    u_ref[...] = jnp.dot(h2, w4p[...], preferred_element_type=jnp.float32)


def _tc_ze_rvq_body(a0, a1, h, dinv, w2, b2, cb, ze_ref, zq_ref, idx_ref, loss_ref):
    i = pl.program_id(0)

    @pl.when(i == 0)
    def _():
        loss_ref[...] = jnp.zeros_like(loss_ref)

    agg = a0[...] + a1[...]
    comb = h[...] + agg * dinv[...]
    z = jnp.dot(comb, w2[...], preferred_element_type=jnp.float32) + b2[...]
    ze_ref[...] = z
    res = z
    zq = jnp.zeros_like(res)
    iota = lax.broadcasted_iota(jnp.int32, (1, K), 1)
    lsum = jnp.float32(0.0)
    idx_cols = []
    for l in range(LEVELS):
        cbl = cb[l]
        cn = jnp.sum(cbl * cbl, axis=1)[None, :]
        rn = jnp.sum(res * res, axis=1, keepdims=True)
        d = (rn - 2.0 * lax.dot_general(res, cbl, (((1,), (1,)), ((), ())),
                                        preferred_element_type=jnp.float32) + cn)
        mv = jnp.min(d, axis=1, keepdims=True)
        idxv = jnp.min(jnp.where(d == mv, iota, K), axis=1, keepdims=True)
        one = (iota == idxv).astype(jnp.float32)
        cb_hi = cbl.astype(jnp.bfloat16).astype(jnp.float32)
        rem1 = cbl - cb_hi
        cb_mid = rem1.astype(jnp.bfloat16).astype(jnp.float32)
        cb_lo = rem1 - cb_mid
        q = (jnp.dot(one, cb_hi, preferred_element_type=jnp.float32)
             + jnp.dot(one, cb_mid, preferred_element_type=jnp.float32)
             + jnp.dot(one, cb_lo, preferred_element_type=jnp.float32))
        lsum = lsum + jnp.sum((res - q) ** 2)
        zq = zq + q
        res = res - q
        idx_cols.append(idxv)
    zq_ref[...] = zq
    pad = jnp.zeros((idxv.shape[0], 8 - LEVELS), jnp.int32)
    idx_ref[...] = jnp.concatenate(idx_cols + [pad], axis=1)
    loss_ref[...] = loss_ref[...] + jnp.reshape(lsum, (1, 1))


def _tc_fin_body(u, a0, a1, dinv, b4p, y, p1m, p2m, r_ref, rl_ref):
    i = pl.program_id(0)

    @pl.when(i == 0)
    def _():
        rl_ref[...] = jnp.zeros_like(rl_ref)

    agg = a0[...] + a1[...]
    r = u[...] + agg * dinv[...] + b4p[...]
    r_ref[...] = r
    t = y[...]
    t1 = jnp.dot(t, p1m[...], preferred_element_type=jnp.float32)
    t2 = jnp.dot(t, p2m[...], preferred_element_type=jnp.float32)
    p0 = jnp.sum(jnp.abs(r - t), axis=1) / 9.0
    p1 = jnp.sum(jnp.abs(r - t1), axis=1) / 9.0
    p2 = jnp.sum(jnp.abs(r - t2), axis=1) / 9.0
    m = jnp.minimum(p0, jnp.minimum(p1, p2))
    rl_ref[...] = rl_ref[...] + jnp.reshape(jnp.sum(m), (1, 1))


def _row_spec(w):
    return pl.BlockSpec((BN, w), lambda i: (i, 0))


def _full_spec(shape):
    nd = len(shape)
    return pl.BlockSpec(shape, lambda i, _nd=nd: (0,) * _nd)


def _scalar_spec():
    return pl.BlockSpec((1, 1), lambda i: (0, 0))


_P1 = np.zeros((CPAD, CPAD), np.float32)
_P2 = np.zeros((CPAD, CPAD), np.float32)
for _j in range(9):
    _P1[(_j + 3) % 9, _j] = 1.0
    _P2[(_j + 6) % 9, _j] = 1.0
for _j in range(9, CPAD):
    _P1[_j, _j] = 1.0
    _P2[_j, _j] = 1.0


def kernel(x, edge_index, y, sv_tri_a, sv_local_a, sv_tri_b, sv_local_b,
           W1, b1, W2, b2, W3, b3, W4, b4, codebooks):
    src = edge_index[0].reshape(NW, NCH, C)
    dst = edge_index[1].reshape(NW, NCH, C)

    x16 = jnp.concatenate(
        [x, jnp.ones((N, 1), jnp.float32), jnp.zeros((N, CPAD - IN_CH - 1), jnp.float32)], axis=1)
    W1p = jnp.concatenate([W1, jnp.zeros((CPAD - IN_CH, LATENT), jnp.float32)], axis=0)
    W4p = jnp.concatenate([W4, jnp.zeros((LATENT, CPAD - IN_CH), jnp.float32)], axis=1)
    b4p = jnp.concatenate([b4, jnp.zeros((CPAD - IN_CH,), jnp.float32)])[None, :]
    y16 = jnp.concatenate([y, jnp.zeros((N, CPAD - IN_CH), jnp.float32)], axis=1)
    b1r = b1[None, :]
    b2r = b2[None, :]
    b3r = b3[None, :]
    p1m = jnp.asarray(_P1)
    p2m = jnp.asarray(_P2)

    agg16 = _make_sc_agg(CPAD)(x16, src, dst)
    a0, a1 = agg16[:N], agg16[NPAD:NPAD + N]

    h, dinv = pl.pallas_call(
        _tc1_body,
        grid=(G,),
        in_specs=[_row_spec(CPAD), _row_spec(CPAD), _row_spec(CPAD),
                  _full_spec((CPAD, LATENT)), _full_spec((1, LATENT))],
        out_specs=[_row_spec(LATENT), _row_spec(1)],
        out_shape=[jax.ShapeDtypeStruct((N, LATENT), jnp.float32),
                   jax.ShapeDtypeStruct((N, 1), jnp.float32)],
    )(a0, a1, x16, W1p, b1r)

    aggh = _make_sc_agg(LATENT)(h, src, dst)

    z_e, z_q, idx8, vqsum = pl.pallas_call(
        _tc_ze_rvq_body,
        grid=(G,),
        in_specs=[_row_spec(LATENT), _row_spec(LATENT), _row_spec(LATENT), _row_spec(1),
                  _full_spec((LATENT, LATENT)), _full_spec((1, LATENT)),
                  _full_spec((LEVELS, K, LATENT))],
        out_specs=[_row_spec(LATENT), _row_spec(LATENT), _row_spec(8), _scalar_spec()],
        out_shape=[jax.ShapeDtypeStruct((N, LATENT), jnp.float32),
                   jax.ShapeDtypeStruct((N, LATENT), jnp.float32),
                   jax.ShapeDtypeStruct((N, 8), jnp.int32),
                   jax.ShapeDtypeStruct((1, 1), jnp.float32)],
    )(aggh[:N], aggh[NPAD:NPAD + N], h, dinv, W2, b2r, codebooks)

    aggq = _make_sc_agg(LATENT)(z_q, src, dst)

    u16 = pl.pallas_call(
        _tc3u_body,
        grid=(G,),
        in_specs=[_row_spec(LATENT), _row_spec(LATENT), _row_spec(LATENT), _row_spec(1),
                  _full_spec((LATENT, LATENT)), _full_spec((1, LATENT)),
                  _full_spec((LATENT, CPAD))],
        out_specs=_row_spec(CPAD),
        out_shape=jax.ShapeDtypeStruct((N, CPAD), jnp.float32),
    )(aggq[:N], aggq[NPAD:NPAD + N], z_q, dinv, W3, b3r, W4p)

    aggu = _make_sc_agg(CPAD)(u16, src, dst)

    r16, rlsum = pl.pallas_call(
        _tc_fin_body,
        grid=(G,),
        in_specs=[_row_spec(CPAD), _row_spec(CPAD), _row_spec(CPAD), _row_spec(1),
                  _full_spec((1, CPAD)), _row_spec(CPAD),
                  _full_spec((CPAD, CPAD)), _full_spec((CPAD, CPAD))],
        out_specs=[_row_spec(CPAD), _scalar_spec()],
        out_shape=[jax.ShapeDtypeStruct((N, CPAD), jnp.float32),
                   jax.ShapeDtypeStruct((1, 1), jnp.float32)],
    )(u16, aggu[:N], aggu[NPAD:NPAD + N], dinv, b4p, y16, p1m, p2m)

    recon = r16[:, :IN_CH]
    rflat = recon.reshape(-1)

    zp = jnp.zeros((PPAD - P,), jnp.int32)
    parts = _sc_cons(rflat,
                     jnp.concatenate([sv_tri_a, zp]).reshape(16, PPW),
                     jnp.concatenate([sv_local_a, zp]).reshape(16, PPW),
                     jnp.concatenate([sv_tri_b, zp]).reshape(16, PPW),
                     jnp.concatenate([sv_local_b, zp]).reshape(16, PPW))

    rl_o, vq_o, c_o, t_o = pl.pallas_call(
        _tc_scalars_body,
        grid=(1,),
        in_specs=[_scalar_spec(), _scalar_spec(), _full_spec((16, 16))],
        out_specs=[_scalar_spec(), _scalar_spec(), _scalar_spec(), _scalar_spec()],
        out_shape=[jax.ShapeDtypeStruct((1, 1), jnp.float32)] * 4,
    )(rlsum, vqsum, parts)

    indices = idx8[:, :LEVELS]
    return (recon, rl_o[0, 0], vq_o[0, 0], c_o[0, 0], t_o[0, 0], indices, z_e, z_q)
